# baseline (speedup 1.0000x reference)
import functools

import jax
import jax.numpy as jnp
from jax import lax
from jax.experimental import pallas as pl
from jax.experimental.pallas import tpu as pltpu
from jax.experimental.pallas import tpu_sc as plsc

NUM_CLASSES = 100000
HIDDEN_SIZE = 64
DROPOUT_PROB = 0.1
BATCH = 16384

NC, NS = 2, 16
NW = NC * NS
B_PER_W = BATCH // NW
CHUNK = 128
NCHUNK = B_PER_W // CHUNK

_mesh = plsc.VectorSubcoreMesh(
    core_axis_name="c", subcore_axis_name="s", num_cores=NC, num_subcores=NS
)


@functools.partial(
    pl.kernel,
    out_type=jax.ShapeDtypeStruct((BATCH, HIDDEN_SIZE), jnp.float32),
    mesh=_mesh,
    compiler_params=pltpu.CompilerParams(use_tc_tiling_on_sc=False),
    scratch_types=[
        pltpu.VMEM((NCHUNK, CHUNK), jnp.int32),
        pltpu.VMEM((B_PER_W, HIDDEN_SIZE), jnp.float32),
        pltpu.SemaphoreType.DMA,
    ],
)
def _gather_rows(labels_hbm, table_hbm, out_hbm, idx_v, rows_v, sem):
    wid = lax.axis_index("s") * NC + lax.axis_index("c")
    base = wid * B_PER_W
    for j in range(NCHUNK):
        pltpu.sync_copy(
            labels_hbm.at[pl.ds(base + j * CHUNK, CHUNK)], idx_v.at[j]
        )
    handles = [
        pltpu.async_copy(
            table_hbm.at[idx_v.at[j]],
            rows_v.at[pl.ds(j * CHUNK, CHUNK)],
            sem,
        )
        for j in range(NCHUNK)
    ]
    for h in handles:
        h.wait()
    pltpu.sync_copy(rows_v, out_hbm.at[pl.ds(base, B_PER_W)])


def kernel(labels, train, embedding_table):
    drop_key = jax.random.key(1)
    drop_ids = jax.random.uniform(drop_key, (labels.shape[0],)) < DROPOUT_PROB
    active = (jnp.asarray(train) != 0) & drop_ids
    labels = jnp.where(active, NUM_CLASSES, labels).astype(jnp.int32)
    return _gather_rows(labels, embedding_table)


# speedup vs baseline: 1.4871x; 1.4871x over previous
import functools

import jax
import jax.numpy as jnp
from jax import lax
from jax.experimental import pallas as pl
from jax.experimental.pallas import tpu as pltpu
from jax.experimental.pallas import tpu_sc as plsc

NUM_CLASSES = 100000
HIDDEN_SIZE = 64
DROPOUT_PROB = 0.1
BATCH = 16384

NC, NS = 2, 16
NW = NC * NS
B_PER_W = BATCH // NW

_mesh = plsc.VectorSubcoreMesh(
    core_axis_name="c", subcore_axis_name="s", num_cores=NC, num_subcores=NS
)


@functools.partial(
    pl.kernel,
    out_type=jax.ShapeDtypeStruct((BATCH, HIDDEN_SIZE), jnp.float32),
    mesh=_mesh,
    scratch_types=[
        pltpu.VMEM((B_PER_W,), jnp.int32),
        pltpu.VMEM((B_PER_W, HIDDEN_SIZE), jnp.float32),
        pltpu.SemaphoreType.DMA,
    ],
)
def _gather_rows(labels_hbm, table_hbm, out_hbm, idx_v, rows_v, sem):
    wid = lax.axis_index("s") * NC + lax.axis_index("c")
    base = wid * B_PER_W
    pltpu.sync_copy(labels_hbm.at[pl.ds(base, B_PER_W)], idx_v)

    def body(k, _):
        chunk = idx_v[pl.ds(k * 16, 16)]
        for j in range(16):
            pltpu.async_copy(
                table_hbm.at[pl.ds(chunk[j], 1)],
                rows_v.at[pl.ds(k * 16 + j, 1)],
                sem,
            )
        return ()

    lax.fori_loop(0, B_PER_W // 16, body, ())
    pltpu.make_async_copy(
        table_hbm.at[pl.ds(0, B_PER_W)], rows_v, sem
    ).wait()
    pltpu.sync_copy(rows_v, out_hbm.at[pl.ds(base, B_PER_W)])


def kernel(labels, train, embedding_table):
    drop_key = jax.random.key(1)
    drop_ids = jax.random.uniform(drop_key, (labels.shape[0],)) < DROPOUT_PROB
    active = (jnp.asarray(train) != 0) & drop_ids
    labels = jnp.where(active, NUM_CLASSES, labels).astype(jnp.int32)
    return _gather_rows(labels, embedding_table)
